# baseline (speedup 1.0000x reference)
import functools

import jax
import jax.numpy as jnp
import numpy as np
from jax import lax
from jax.experimental import pallas as pl
from jax.experimental.pallas import tpu as pltpu
from jax.experimental.pallas import tpu_sc as plsc

NC, NS = 2, 16
NW = NC * NS
B, H, D = 4096, 50, 64
BPW = B // NW
G = 2
CH = BPW // G
GH = 104
NBUF = 4
LANES = 16
NI = D // 32

_mesh = plsc.VectorSubcoreMesh(core_axis_name="c", subcore_axis_name="s",
                               num_cores=NC, num_subcores=NS)

_PERM = np.empty(D, dtype=np.int32)
for _k in range(NI):
    for _j in range(32):
        _PERM[32 * _k + _j] = 32 * _k + (_j // 2 + 16 * (_j % 2))


@functools.partial(
    pl.kernel,
    out_type=jax.ShapeDtypeStruct((B, D), jnp.float32),
    mesh=_mesh,
    compiler_params=pltpu.CompilerParams(use_tc_tiling_on_sc=False,
                                         needs_layout_passes=False),
    scratch_types=[
        pltpu.VMEM((CH, GH), jnp.int32),
        [pltpu.VMEM((GH, D), jnp.bfloat16) for _ in range(NBUF)],
        pltpu.VMEM((BPW, D), jnp.float32),
        [pltpu.SemaphoreType.DMA for _ in range(NBUF)],
    ],
)
def _sc_embed_sum(idx_hbm, table_hbm, out_hbm, idx_v, bufs, out_v, sems):
    wid = lax.axis_index("s") * NC + lax.axis_index("c")
    pltpu.sync_copy(idx_hbm.at[wid], idx_v)

    def start(g, b):
        pltpu.async_copy(table_hbm.at[idx_v.at[g]], bufs[b], sems[b])

    def wait(b):
        pltpu.make_async_copy(table_hbm.at[pl.ds(0, GH)], bufs[b],
                              sems[b]).wait()

    hi_mask = jnp.full((LANES,), np.int32(np.uint32(0xFFFF0000).view(np.int32)),
                       dtype=jnp.int32)

    def widen(v32):
        w = plsc.bitcast(v32, jnp.int32)
        lo = plsc.bitcast(lax.shift_left(w, 16), jnp.float32)
        hi = plsc.bitcast(lax.bitwise_and(w, hi_mask), jnp.float32)
        return lo, hi

    def accum(buf, g):
        for e in range(G):
            accs = None
            for r in range(H):
                vals = []
                for k in range(NI):
                    lo, hi = widen(buf[e * H + r, pl.ds(32 * k, 32)])
                    vals += [lo, hi]
                if accs is None:
                    accs = vals
                else:
                    accs = [a + v for a, v in zip(accs, vals)]
            for k in range(NI):
                out_v[g * G + e, pl.ds(32 * k, LANES)] = accs[2 * k]
                out_v[g * G + e, pl.ds(32 * k + LANES, LANES)] = accs[2 * k + 1]

    for b in range(NBUF - 1):
        start(b, b)

    def body(i, carry):
        g0 = NBUF * i
        for b in range(NBUF):
            g = g0 + b
            nb = (b + NBUF - 1) % NBUF

            @pl.when(g + NBUF - 1 < CH)
            def _():
                start(g + NBUF - 1, nb)

            wait(b)
            accum(bufs[b], g)
        return carry

    lax.fori_loop(0, CH // NBUF, body, 0)
    pltpu.sync_copy(out_v, out_hbm.at[pl.ds(wid * BPW, BPW)])


V = 1000000
TCH = 512


def _tc_body(tT_ref, out_ref):
    out_ref[...] = jnp.transpose(tT_ref[...], (1, 0)).astype(jnp.bfloat16)


_tc_transpose = pl.pallas_call(
    _tc_body,
    grid=((V + TCH - 1) // TCH,),
    in_specs=[pl.BlockSpec((D, TCH), lambda j: (0, j))],
    out_specs=pl.BlockSpec((TCH, D), lambda j: (j, 0)),
    out_shape=jax.ShapeDtypeStruct((V, D), jnp.bfloat16),
)


def kernel(query, table):
    q = query.reshape(NW, CH, G * H)
    q = jnp.pad(q, ((0, 0), (0, 0), (0, GH - G * H)))
    out = _sc_embed_sum(q, _tc_transpose(table.T))
    out = jnp.take(out, jnp.asarray(_PERM), axis=1)
    return out[:, None, :]


# speedup vs baseline: 2.2337x; 2.2337x over previous
import functools

import jax
import jax.numpy as jnp
from jax import lax
from jax.experimental import pallas as pl
from jax.experimental.pallas import tpu as pltpu
from jax.experimental.pallas import tpu_sc as plsc

NC, NS = 2, 16
NW = NC * NS
B, H, D = 4096, 50, 64
HB = B // 2
CPW = HB // NW
GH = 104
QW = 128
LANES = 16
LG = D // LANES

_mesh = plsc.VectorSubcoreMesh(core_axis_name="c", subcore_axis_name="s",
                               num_cores=NC, num_subcores=NS)


@functools.partial(
    pl.kernel,
    out_type=jax.ShapeDtypeStruct((B, D), jnp.float32),
    mesh=_mesh,
    compiler_params=pltpu.CompilerParams(use_tc_tiling_on_sc=False,
                                         needs_layout_passes=False),
    scratch_types=[
        pltpu.VMEM((CPW, QW), jnp.int32),
        pltpu.VMEM((GH, D), jnp.float32),
        pltpu.VMEM((GH, D), jnp.float32),
        pltpu.VMEM((CPW, D), jnp.float32),
        pltpu.VMEM((CPW, D), jnp.float32),
        pltpu.SemaphoreType.DMA,
        pltpu.SemaphoreType.DMA,
    ],
)
def _sc_embed_sum(q_hbm, table_hbm, out_hbm, idx_v, buf0, buf1, outa_v,
                  outb_v, sem0, sem1):
    wid = lax.axis_index("s") * NC + lax.axis_index("c")
    pltpu.sync_copy(q_hbm.at[wid], idx_v)

    def start(g, buf, sem):
        pltpu.async_copy(table_hbm.at[idx_v.at[g, pl.ds(0, GH)]], buf, sem)

    def wait(buf, sem):
        pltpu.make_async_copy(table_hbm.at[pl.ds(0, GH)], buf, sem).wait()

    def accum(buf, g):
        for half, out_v in ((0, outa_v), (1, outb_v)):
            for l in range(LG):
                acc = buf[half * H, pl.ds(l * LANES, LANES)]
                for r in range(1, H):
                    acc = acc + buf[half * H + r, pl.ds(l * LANES, LANES)]
                out_v[g, pl.ds(l * LANES, LANES)] = acc

    start(0, buf0, sem0)

    def body(i, carry):
        g = 2 * i
        start(g + 1, buf1, sem1)
        wait(buf0, sem0)
        accum(buf0, g)

        @pl.when(g + 2 < CPW)
        def _():
            start(g + 2, buf0, sem0)

        wait(buf1, sem1)
        accum(buf1, g + 1)
        return carry

    lax.fori_loop(0, CPW // 2, body, 0)
    pltpu.sync_copy(outa_v, out_hbm.at[pl.ds(wid * CPW, CPW)])
    pltpu.sync_copy(outb_v, out_hbm.at[pl.ds(HB + wid * CPW, CPW)])


QBLK = 128


def _tc_fmt_body(qa_ref, qb_ref, out_ref):
    pad = jnp.zeros((QBLK, QW - 2 * H), jnp.int32)
    out_ref[...] = jnp.concatenate([qa_ref[...], qb_ref[...], pad], axis=1)


_tc_fmt = pl.pallas_call(
    _tc_fmt_body,
    grid=(HB // QBLK,),
    in_specs=[
        pl.BlockSpec((QBLK, H), lambda j: (j, 0)),
        pl.BlockSpec((QBLK, H), lambda j: (j + HB // QBLK, 0)),
    ],
    out_specs=pl.BlockSpec((QBLK, QW), lambda j: (j, 0)),
    out_shape=jax.ShapeDtypeStruct((HB, QW), jnp.int32),
)


def kernel(query, table):
    q = _tc_fmt(query, query).reshape(NW, CPW, QW)
    out = _sc_embed_sum(q, table)
    return out[:, None, :]
